# baseline (speedup 1.0000x reference)
import jax
import jax.numpy as jnp
from jax.experimental import pallas as pl
from jax.experimental.pallas import tpu as pltpu

_M = 1048576
_B = 2048


def _body(x_ref, a_ref, index_ref, o_ref):
    pid = pl.program_id(0)
    base = pid * _B
    yt = x_ref[...].T
    rows = jax.lax.broadcasted_iota(jnp.int32, (_B, 1), 0) + base
    acc = yt
    for k in range(3):
        r = index_ref[k]
        acc = acc + jnp.where(rows == r, a_ref[k:k + 1, :], 0.0)
    o_ref[...] = acc


def kernel(x, a, index):
    grid = (_M // _B,)
    return pl.pallas_call(
        _body,
        grid=grid,
        in_specs=[
            pl.BlockSpec((3, _B), lambda i: (0, i)),
            pl.BlockSpec((3, 3), lambda i: (0, 0)),
            pl.BlockSpec(memory_space=pltpu.SMEM),
        ],
        out_specs=pl.BlockSpec((_B, 3), lambda i: (i, 0)),
        out_shape=jax.ShapeDtypeStruct((_M, 3), jnp.float32),
    )(x, a, index.astype(jnp.int32))
